# baseline (speedup 1.0000x reference)
import functools

import jax
import jax.numpy as jnp
from jax import lax
from jax.experimental import pallas as pl
from jax.experimental.pallas import tpu as pltpu
from jax.experimental.pallas import tpu_sc as plsc

B = 1024
D = 32
N = 100000
K = 16
H = 24
F = 7
R = 128
LBLK = 2048
NBLK = 49
NPAD = NBLK * LBLK
G = NPAD // R
GPB = LBLK // R
NBLK_A = 25
NBLK_B = NBLK - NBLK_A
GA = NBLK_A * GPB
GB = NBLK_B * GPB
NEG = float("-inf")
DECAY = 0.995
STEP = 1000.0

NC = 2
NS = 16
NW = NC * NS
ROWS_W = (B * K) // NW
CHUNK = 128
VW = 256
NCH = ROWS_W // CHUNK


def _sims_groups_body(qn_ref, kn_ref, decay_ref, sims_ref, gmax_ref, *, off):
    i = pl.program_id(0) + off

    sims = lax.dot_general(qn_ref[...], kn_ref[...], (((1,), (1,)), ((), ())),
                           preferred_element_type=jnp.float32)
    sims = sims * decay_ref[...][None, :]

    lane = i * LBLK + lax.broadcasted_iota(jnp.int32, (1, LBLK), 1)
    sims = jnp.where(lane < N, sims, NEG)

    for g in range(GPB):
        sims_ref[g] = sims[:, g * R:(g + 1) * R]

    gmax_ref[...] = jnp.max(sims.reshape(B, GPB, R), axis=2)[None]


def _group_topk_body(gmax_ref, gids_ref, flata_ref, flatb_ref):
    S = gmax_ref[...]
    giota = lax.broadcasted_iota(jnp.int32, (B, G), 1)
    picks = []
    for _ in range(K):
        m = jnp.max(S, axis=1, keepdims=True)
        am = jnp.min(jnp.where(S == m, giota, G), axis=1, keepdims=True)
        picks.append(am)
        S = jnp.where(giota == am, NEG, S)
    gids = jnp.concatenate(picks, axis=1)
    gids_ref[...] = gids
    row = lax.broadcasted_iota(jnp.int32, (B, K), 0)
    flata_ref[...] = jnp.minimum(gids, GA - 1) * B + row
    flatb_ref[...] = jnp.clip(gids - GA, 0, GB - 1) * B + row


def _final_body(canda_ref, candb_ref, gids_ref, sims_out_ref, idx_out_ref,
                mask_out_ref):
    gids = gids_ref[...]
    glane = jnp.reshape(
        jnp.broadcast_to(gids[:, :, None], (B, K, R)), (B, K * R))
    S = jnp.where(glane < GA, canda_ref[...], candb_ref[...])
    ciota = lax.broadcasted_iota(jnp.int32, (B, K * R), 1)
    vals, ams = [], []
    for _ in range(K):
        m = jnp.max(S, axis=1, keepdims=True)
        am = jnp.min(jnp.where(S == m, ciota, K * R), axis=1, keepdims=True)
        vals.append(m)
        ams.append(am)
        S = jnp.where(ciota == am, NEG, S)
    top = jnp.concatenate(vals, axis=1)
    am = jnp.concatenate(ams, axis=1)
    slot = am // R
    pos = am % R
    gsel = jnp.zeros((B, K), jnp.int32)
    for j in range(K):
        gsel = jnp.where(slot == j, gids[:, j:j + 1], gsel)
    sims_out_ref[...] = top
    idx_out_ref[...] = gsel * R + pos
    mask_out_ref[...] = top >= 0.0


@functools.lru_cache(maxsize=None)
def _make_sc_gather(rows, row_shape, tc_tiling=True):
    row_shape = (row_shape,) if isinstance(row_shape, int) else tuple(row_shape)
    mesh = plsc.VectorSubcoreMesh(core_axis_name="c", subcore_axis_name="s",
                                  num_cores=NC, num_subcores=NS)
    rows_w = rows // NW
    nch = rows_w // CHUNK

    @functools.partial(
        pl.kernel,
        out_type=jax.ShapeDtypeStruct((rows,) + row_shape, jnp.float32),
        mesh=mesh,
        scratch_types=[
            pltpu.VMEM((nch, CHUNK), jnp.int32),
            pltpu.VMEM((2, CHUNK) + row_shape, jnp.float32),
            pltpu.SemaphoreType.DMA,
        ],
        compiler_params=pltpu.CompilerParams(use_tc_tiling_on_sc=tc_tiling),
    )
    def gather(table_hbm, idx_hbm, out_hbm, idx_v, rows_v, sem):
        wid = lax.axis_index("s") * NC + lax.axis_index("c")
        pltpu.sync_copy(idx_hbm.at[pl.ds(wid * nch, nch)], idx_v)
        copies = [None] * nch
        for c in range(min(2, nch)):
            copies[c] = pltpu.async_copy(table_hbm.at[idx_v.at[c]],
                                         rows_v.at[c % 2], sem)
        for c in range(nch):
            copies[c].wait()
            pltpu.sync_copy(
                rows_v.at[c % 2],
                out_hbm.at[pl.ds(wid * rows_w + c * CHUNK, CHUNK)])
            if c + 2 < nch:
                copies[c + 2] = pltpu.async_copy(
                    table_hbm.at[idx_v.at[c + 2]], rows_v.at[c % 2], sem)

    return gather


def _l2n(x, eps=1e-12):
    n = jnp.linalg.norm(x, axis=-1, keepdims=True)
    return x / jnp.maximum(n, eps)


def kernel(query, keys, values, timestamps, top_k):
    del top_k
    qn = _l2n(query)
    kn = _l2n(keys)
    age = (1000 - timestamps).astype(jnp.float32)
    decay = jnp.power(jnp.float32(DECAY), age)

    def run_half(nblk, off):
        return pl.pallas_call(
            functools.partial(_sims_groups_body, off=off),
            grid=(nblk,),
            in_specs=[
                pl.BlockSpec((B, D), lambda i: (0, 0)),
                pl.BlockSpec((LBLK, D), lambda i: (i + off, 0)),
                pl.BlockSpec((LBLK,), lambda i: (i + off,)),
            ],
            name=f"sims_groupmax_{off}",
            out_specs=[
                pl.BlockSpec((GPB, B, R), lambda i: (i, 0, 0)),
                pl.BlockSpec((1, B, GPB), lambda i: (i, 0, 0)),
            ],
            out_shape=[
                jax.ShapeDtypeStruct((nblk * GPB, B, R), jnp.float32),
                jax.ShapeDtypeStruct((nblk, B, GPB), jnp.float32),
            ],
        )(qn, kn, decay)

    sims_a, gmax_a = run_half(NBLK_A, 0)
    sims_b, gmax_b = run_half(NBLK_B, NBLK_A)
    gmax = jnp.concatenate([gmax_a, gmax_b], axis=0)

    gids, flat_a, flat_b = pl.pallas_call(
        _group_topk_body,
        out_shape=[
            jax.ShapeDtypeStruct((B, K), jnp.int32),
            jax.ShapeDtypeStruct((B, K), jnp.int32),
            jax.ShapeDtypeStruct((B, K), jnp.int32),
        ],
    )(gmax.transpose(1, 0, 2).reshape(B, G))

    cand_a = _make_sc_gather(B * K, R)(
        sims_a.reshape(GA * B, R), flat_a.reshape((B * K) // CHUNK, CHUNK))
    cand_b = _make_sc_gather(B * K, R)(
        sims_b.reshape(GB * B, R), flat_b.reshape((B * K) // CHUNK, CHUNK))

    top_sims, top_idx, mask = pl.pallas_call(
        _final_body,
        out_shape=[
            jax.ShapeDtypeStruct((B, K), jnp.float32),
            jax.ShapeDtypeStruct((B, K), jnp.int32),
            jax.ShapeDtypeStruct((B, K), jnp.bool_),
        ],
    )(cand_a.reshape(B, K * R), cand_b.reshape(B, K * R), gids)

    vpad = jnp.pad(values.reshape(N, H * F), ((0, 0), (0, VW - H * F)))
    retrieved = _make_sc_gather(B * K, VW)(
        vpad, top_idx.reshape((B * K) // CHUNK, CHUNK))

    return retrieved[:, :H * F].reshape(B, K, H, F), top_sims, mask


# speedup vs baseline: 1.0378x; 1.0378x over previous
import functools

import jax
import jax.numpy as jnp
from jax import lax
from jax.experimental import pallas as pl
from jax.experimental.pallas import tpu as pltpu
from jax.experimental.pallas import tpu_sc as plsc

B = 1024
D = 32
N = 100000
K = 16
H = 24
F = 7
R = 128
LBLK = 4096
NBLK = 25
NPAD = NBLK * LBLK
G = NPAD // R
GPB = LBLK // R
NEG = float("-inf")
DECAY = 0.995
STEP = 1000.0

NC = 2
NS = 16
NW = NC * NS
ROWS_W = (B * K) // NW
CHUNK = 128
VW = 256
NCH = ROWS_W // CHUNK


def _sims_groups_body(qn_ref, kn_ref, decay_ref, sims_ref, gmax_ref, *, off):
    i = pl.program_id(0) + off

    sims = lax.dot_general(qn_ref[...], kn_ref[...], (((1,), (1,)), ((), ())),
                           preferred_element_type=jnp.float32)
    sims = sims * decay_ref[...][None, :]

    lane = i * LBLK + lax.broadcasted_iota(jnp.int32, (1, LBLK), 1)
    sims = jnp.where(lane < N, sims, NEG)

    for g in range(GPB):
        sims_ref[g] = sims[:, g * R:(g + 1) * R]

    gmax_ref[...] = jnp.max(sims.reshape(B, GPB, R), axis=2)[None]


def _group_topk_body(gmax_ref, gids_ref, flat_ref):
    S = gmax_ref[...]
    giota = lax.broadcasted_iota(jnp.int32, (B, G), 1)
    picks = []
    for _ in range(K):
        m = jnp.max(S, axis=1, keepdims=True)
        am = jnp.min(jnp.where(S == m, giota, G), axis=1, keepdims=True)
        picks.append(am)
        S = jnp.where(giota == am, NEG, S)
    gids = jnp.concatenate(picks, axis=1)
    gids_ref[...] = gids
    row = lax.broadcasted_iota(jnp.int32, (B, K), 0)
    flat_ref[...] = gids * B + row


def _final_body(cand_ref, gids_ref, sims_out_ref, idx_out_ref, mask_out_ref):
    gids = gids_ref[...]
    S = cand_ref[...]
    ciota = lax.broadcasted_iota(jnp.int32, (B, K * R), 1)
    vals, ams = [], []
    for _ in range(K):
        m = jnp.max(S, axis=1, keepdims=True)
        am = jnp.min(jnp.where(S == m, ciota, K * R), axis=1, keepdims=True)
        vals.append(m)
        ams.append(am)
        S = jnp.where(ciota == am, NEG, S)
    top = jnp.concatenate(vals, axis=1)
    am = jnp.concatenate(ams, axis=1)
    slot = am // R
    pos = am % R
    gsel = jnp.zeros((B, K), jnp.int32)
    for j in range(K):
        gsel = jnp.where(slot == j, gids[:, j:j + 1], gsel)
    sims_out_ref[...] = top
    idx_out_ref[...] = gsel * R + pos
    mask_out_ref[...] = top >= 0.0


@functools.lru_cache(maxsize=None)
def _make_sc_gather(rows, row_shape, tc_tiling=True):
    row_shape = (row_shape,) if isinstance(row_shape, int) else tuple(row_shape)
    mesh = plsc.VectorSubcoreMesh(core_axis_name="c", subcore_axis_name="s",
                                  num_cores=NC, num_subcores=NS)
    rows_w = rows // NW
    nch = rows_w // CHUNK

    @functools.partial(
        pl.kernel,
        out_type=jax.ShapeDtypeStruct((rows,) + row_shape, jnp.float32),
        mesh=mesh,
        scratch_types=[
            pltpu.VMEM((nch, CHUNK), jnp.int32),
            pltpu.VMEM((2, CHUNK) + row_shape, jnp.float32),
            pltpu.SemaphoreType.DMA,
        ],
        compiler_params=pltpu.CompilerParams(use_tc_tiling_on_sc=tc_tiling),
    )
    def gather(table_hbm, idx_hbm, out_hbm, idx_v, rows_v, sem):
        wid = lax.axis_index("s") * NC + lax.axis_index("c")
        pltpu.sync_copy(idx_hbm.at[pl.ds(wid * nch, nch)], idx_v)
        copies = [None] * nch
        for c in range(min(2, nch)):
            copies[c] = pltpu.async_copy(table_hbm.at[idx_v.at[c]],
                                         rows_v.at[c % 2], sem)
        for c in range(nch):
            copies[c].wait()
            pltpu.sync_copy(
                rows_v.at[c % 2],
                out_hbm.at[pl.ds(wid * rows_w + c * CHUNK, CHUNK)])
            if c + 2 < nch:
                copies[c + 2] = pltpu.async_copy(
                    table_hbm.at[idx_v.at[c + 2]], rows_v.at[c % 2], sem)

    return gather


def _l2n(x, eps=1e-12):
    n = jnp.linalg.norm(x, axis=-1, keepdims=True)
    return x / jnp.maximum(n, eps)


def kernel(query, keys, values, timestamps, top_k):
    del top_k
    qn = _l2n(query)
    kn = _l2n(keys)
    age = (1000 - timestamps).astype(jnp.float32)
    decay = jnp.power(jnp.float32(DECAY), age)

    def run_half(nblk, off):
        return pl.pallas_call(
            functools.partial(_sims_groups_body, off=off),
            grid=(nblk,),
            in_specs=[
                pl.BlockSpec((B, D), lambda i: (0, 0)),
                pl.BlockSpec((LBLK, D), lambda i: (i + off, 0)),
                pl.BlockSpec((LBLK,), lambda i: (i + off,)),
            ],
            name=f"sims_groupmax_{off}",
            out_specs=[
                pl.BlockSpec((GPB, B, R), lambda i: (i, 0, 0)),
                pl.BlockSpec((1, B, GPB), lambda i: (i, 0, 0)),
            ],
            out_shape=[
                jax.ShapeDtypeStruct((nblk * GPB, B, R), jnp.float32),
                jax.ShapeDtypeStruct((nblk, B, GPB), jnp.float32),
            ],
        )(qn, kn, decay)

    sims, gmax = run_half(NBLK, 0)

    gids, flat = pl.pallas_call(
        _group_topk_body,
        out_shape=[
            jax.ShapeDtypeStruct((B, K), jnp.int32),
            jax.ShapeDtypeStruct((B, K), jnp.int32),
        ],
    )(gmax.transpose(1, 0, 2).reshape(B, G))

    cand = _make_sc_gather(B * K, R)(
        sims.reshape(G * B, R), flat.reshape((B * K) // CHUNK, CHUNK))

    top_sims, top_idx, mask = pl.pallas_call(
        _final_body,
        out_shape=[
            jax.ShapeDtypeStruct((B, K), jnp.float32),
            jax.ShapeDtypeStruct((B, K), jnp.int32),
            jax.ShapeDtypeStruct((B, K), jnp.bool_),
        ],
    )(cand.reshape(B, K * R), gids)

    vpad = jnp.pad(values.reshape(N, H * F), ((0, 0), (0, VW - H * F)))
    retrieved = _make_sc_gather(B * K, VW)(
        vpad, top_idx.reshape((B * K) // CHUNK, CHUNK))

    return retrieved[:, :H * F].reshape(B, K, H, F), top_sims, mask


# speedup vs baseline: 1.0381x; 1.0003x over previous
import functools

import jax
import jax.numpy as jnp
from jax import lax
from jax.experimental import pallas as pl
from jax.experimental.pallas import tpu as pltpu
from jax.experimental.pallas import tpu_sc as plsc

B = 1024
D = 32
N = 100000
K = 16
H = 24
F = 7
R = 128
LBLK = 4096
NBLK = 25
NPAD = NBLK * LBLK
G = NPAD // R
GPB = LBLK // R
NEG = float("-inf")
DECAY = 0.995

NC = 2
NS = 16
NW = NC * NS
CHUNK = 128
VW = 256


def _sims_groups_body(qn_ref, kn_ref, decay_ref, sims_ref, gmax_ref, *, off):
    i = pl.program_id(0) + off

    sims = lax.dot_general(qn_ref[...], kn_ref[...], (((1,), (1,)), ((), ())),
                           preferred_element_type=jnp.float32)
    sims = sims * decay_ref[...][None, :]

    lane = i * LBLK + lax.broadcasted_iota(jnp.int32, (1, LBLK), 1)
    sims = jnp.where(lane < N, sims, NEG)

    for g in range(GPB):
        sims_ref[g] = sims[:, g * R:(g + 1) * R]

    gmax_ref[...] = jnp.max(sims.reshape(B, GPB, R), axis=2)[None]


def _group_topk_body(gmax_ref, gids_ref, flat_ref):
    S = gmax_ref[...]
    giota = lax.broadcasted_iota(jnp.int32, (B, G), 1)
    picks = []
    for _ in range(K):
        m = jnp.max(S, axis=1, keepdims=True)
        am = jnp.min(jnp.where(S == m, giota, G), axis=1, keepdims=True)
        picks.append(am)
        S = jnp.where(giota == am, NEG, S)
    gids = jnp.concatenate(picks, axis=1)
    gids_ref[...] = gids
    row = lax.broadcasted_iota(jnp.int32, (B, K), 0)
    flat_ref[...] = gids * B + row


def _final_body(cand_ref, gids_ref, sims_out_ref, idx_out_ref, mask_out_ref):
    gids = gids_ref[...]
    S = cand_ref[...]
    ciota = lax.broadcasted_iota(jnp.int32, (B, K * R), 1)
    vals, ams = [], []
    for _ in range(K):
        m = jnp.max(S, axis=1, keepdims=True)
        am = jnp.min(jnp.where(S == m, ciota, K * R), axis=1, keepdims=True)
        vals.append(m)
        ams.append(am)
        S = jnp.where(ciota == am, NEG, S)
    top = jnp.concatenate(vals, axis=1)
    am = jnp.concatenate(ams, axis=1)
    slot = am // R
    pos = am % R
    gsel = jnp.zeros((B, K), jnp.int32)
    for j in range(K):
        gsel = jnp.where(slot == j, gids[:, j:j + 1], gsel)
    sims_out_ref[...] = top
    idx_out_ref[...] = gsel * R + pos
    mask_out_ref[...] = top >= 0.0


@functools.lru_cache(maxsize=None)
def _make_sc_gather(rows, row_shape, tc_tiling=True):
    row_shape = (row_shape,) if isinstance(row_shape, int) else tuple(row_shape)
    mesh = plsc.VectorSubcoreMesh(core_axis_name="c", subcore_axis_name="s",
                                  num_cores=NC, num_subcores=NS)
    rows_w = rows // NW
    nch = rows_w // CHUNK

    @functools.partial(
        pl.kernel,
        out_type=jax.ShapeDtypeStruct((rows,) + row_shape, jnp.float32),
        mesh=mesh,
        scratch_types=[
            pltpu.VMEM((nch, CHUNK), jnp.int32),
            pltpu.VMEM((2, CHUNK) + row_shape, jnp.float32),
            pltpu.SemaphoreType.DMA,
        ],
        compiler_params=pltpu.CompilerParams(use_tc_tiling_on_sc=tc_tiling),
    )
    def gather(table_hbm, idx_hbm, out_hbm, idx_v, rows_v, sem):
        wid = lax.axis_index("s") * NC + lax.axis_index("c")
        pltpu.sync_copy(idx_hbm.at[pl.ds(wid * nch, nch)], idx_v)
        copies = [None] * nch
        for c in range(min(2, nch)):
            copies[c] = pltpu.async_copy(table_hbm.at[idx_v.at[c]],
                                         rows_v.at[c % 2], sem)
        for c in range(nch):
            copies[c].wait()
            pltpu.sync_copy(
                rows_v.at[c % 2],
                out_hbm.at[pl.ds(wid * rows_w + c * CHUNK, CHUNK)])
            if c + 2 < nch:
                copies[c + 2] = pltpu.async_copy(
                    table_hbm.at[idx_v.at[c + 2]], rows_v.at[c % 2], sem)

    return gather


def _l2n(x, eps=1e-12):
    n = jnp.linalg.norm(x, axis=-1, keepdims=True)
    return x / jnp.maximum(n, eps)


def kernel(query, keys, values, timestamps, top_k):
    del top_k
    qn = _l2n(query)
    kn = _l2n(keys)
    age = (1000 - timestamps).astype(jnp.float32)
    decay = jnp.power(jnp.float32(DECAY), age)

    def run_half(nblk, off):
        return pl.pallas_call(
            functools.partial(_sims_groups_body, off=off),
            grid=(nblk,),
            in_specs=[
                pl.BlockSpec((B, D), lambda i: (0, 0)),
                pl.BlockSpec((LBLK, D), lambda i: (i + off, 0)),
                pl.BlockSpec((LBLK,), lambda i: (i + off,)),
            ],
            name=f"sims_groupmax_{off}",
            out_specs=[
                pl.BlockSpec((GPB, B, R), lambda i: (i, 0, 0)),
                pl.BlockSpec((1, B, GPB), lambda i: (i, 0, 0)),
            ],
            out_shape=[
                jax.ShapeDtypeStruct((nblk * GPB, B, R), jnp.float32),
                jax.ShapeDtypeStruct((nblk, B, GPB), jnp.float32),
            ],
        )(qn, kn, decay)

    sims, gmax = run_half(NBLK, 0)

    gids, flat = pl.pallas_call(
        _group_topk_body,
        out_shape=[
            jax.ShapeDtypeStruct((B, K), jnp.int32),
            jax.ShapeDtypeStruct((B, K), jnp.int32),
        ],
    )(gmax.transpose(1, 0, 2).reshape(B, G))

    cand = _make_sc_gather(B * K, R)(
        sims.reshape(G * B, R), flat.reshape((B * K) // CHUNK, CHUNK))

    top_sims, top_idx, mask = pl.pallas_call(
        _final_body,
        out_shape=[
            jax.ShapeDtypeStruct((B, K), jnp.float32),
            jax.ShapeDtypeStruct((B, K), jnp.int32),
            jax.ShapeDtypeStruct((B, K), jnp.bool_),
        ],
    )(cand.reshape(B, K * R), gids)

    vpad = jnp.pad(values.reshape(N, H * F), ((0, 0), (0, VW - H * F)))
    retrieved = _make_sc_gather(B * K, VW)(
        vpad, top_idx.reshape((B * K) // CHUNK, CHUNK))

    return retrieved[:, :H * F].reshape(B, K, H, F), top_sims, mask
